# baseline (speedup 1.0000x reference)
import functools

import jax
import jax.numpy as jnp
from jax import lax
from jax.experimental import pallas as pl
from jax.experimental.pallas import tpu as pltpu
from jax.experimental.pallas import tpu_sc as plsc

B, T, N, F_DYN, F_STA = 2, 12, 10000, 16, 16
H, G, E = 64, 64, 320000

NTILE = 16
NCORE = 2
EPAD = 327680
UNIT = 512
UPT = EPAD // (NTILE * UNIT)
NP = 10240
RPT = NP // NTILE
GP = 64
FW = T * F_DYN + F_STA


def _vmesh():
    return plsc.VectorSubcoreMesh(core_axis_name="c", subcore_axis_name="s")


def _zero_vmem_2d(buf, rows, cols):
    zv = jnp.zeros((16,), jnp.float32)

    @pl.loop(0, rows)
    def _(r):
        @pl.loop(0, cols // 16)
        def _(j):
            buf[r, pl.ds(j * 16, 16)] = zv


def _zero_vmem_1d(buf, n):
    zv = jnp.zeros((16,), jnp.float32)

    @pl.loop(0, n // 16)
    def _(j):
        buf[pl.ds(j * 16, 16)] = zv


def _k0_counts(dst2d):
    @functools.partial(
        pl.kernel,
        mesh=_vmesh(),
        compiler_params=pltpu.CompilerParams(use_tc_tiling_on_sc=False),
        out_type=jax.ShapeDtypeStruct((NP,), jnp.float32),
        scratch_types=[
            pltpu.VMEM((4, 128), jnp.int32),
            pltpu.VMEM((128,), jnp.float32),
            pltpu.VMEM((RPT,), jnp.float32),
            pltpu.VMEM_SHARED((NP,), jnp.float32),
        ],
    )
    def k0(dst_hbm, out_hbm, didx, ones_v, zbuf, cnt_sh):
        c = lax.axis_index("c")
        s = lax.axis_index("s")

        @pl.when(c == 0)
        def _():
            ov = jnp.ones((16,), jnp.float32)

            @pl.loop(0, 8)
            def _(j):
                ones_v[pl.ds(j * 16, 16)] = ov

            _zero_vmem_1d(zbuf, RPT)
            pltpu.sync_copy(zbuf, cnt_sh.at[pl.ds(s * RPT, RPT)])
            plsc.subcore_barrier()

            row0 = s * (UPT * 4)

            @pl.loop(0, UPT)
            def _(u):
                pltpu.sync_copy(dst_hbm.at[pl.ds(row0 + u * 4, 4)], didx)
                for j in range(4):
                    pltpu.sync_copy(ones_v, cnt_sh.at[didx.at[j]], add=True)

            plsc.subcore_barrier()
            pltpu.sync_copy(cnt_sh.at[pl.ds(s * RPT, RPT)],
                            out_hbm.at[pl.ds(s * RPT, RPT)])

    return k0(dst2d)


def _k2_seg64(x, src3, dst2d):
    @functools.partial(
        pl.kernel,
        mesh=_vmesh(),
        compiler_params=pltpu.CompilerParams(use_tc_tiling_on_sc=False),
        out_type=jax.ShapeDtypeStruct((B * N, GP), jnp.float32),
        scratch_types=[
            pltpu.VMEM((2, 4, 128), jnp.int32),
            pltpu.VMEM((2, 4, 128), jnp.int32),
            pltpu.VMEM((2, UNIT, GP), jnp.float32),
            pltpu.VMEM((128, GP), jnp.float32),
            pltpu.VMEM_SHARED((NP, GP), jnp.float32),
            pltpu.SemaphoreType.DMA,
            pltpu.SemaphoreType.DMA,
            pltpu.SemaphoreType.DMA,
            pltpu.SemaphoreType.DMA,
            pltpu.SemaphoreType.DMA,
        ],
    )
    def k2(x_hbm, src_hbm, dst_hbm, out_hbm, sidx, didx, vals, zbuf, acc_sh,
           semi, semg0, semg1, sems0, sems1):
        c = lax.axis_index("c")
        s = lax.axis_index("s")
        semg = (semg0, semg1)
        sems = (sems0, sems1)

        _zero_vmem_2d(zbuf, 128, GP)

        @pl.loop(0, RPT // 128)
        def _(k):
            pltpu.sync_copy(zbuf, acc_sh.at[pl.ds(s * RPT + k * 128, 128)])

        plsc.subcore_barrier()

        row0 = s * (UPT * 4)

        def load_idx_sync(slot, u):
            r = row0 + u * 4
            h1 = pltpu.async_copy(src_hbm.at[c, pl.ds(r, 4)], sidx.at[slot],
                                  semi)
            h2 = pltpu.async_copy(dst_hbm.at[pl.ds(r, 4)], didx.at[slot], semi)
            h1.wait()
            h2.wait()

        def fire_g(slot):
            for j in range(4):
                pltpu.async_copy(x_hbm.at[sidx.at[slot, j]],
                                 vals.at[slot, pl.ds(j * 128, 128)],
                                 semg[slot])

        def wait_g(slot):
            for j in range(4):
                pltpu.make_async_copy(x_hbm.at[pl.ds(0, 128)],
                                      vals.at[slot, pl.ds(j * 128, 128)],
                                      semg[slot]).wait()

        def fire_s(slot):
            for j in range(4):
                pltpu.async_copy(vals.at[slot, pl.ds(j * 128, 128)],
                                 acc_sh.at[didx.at[slot, j]], sems[slot],
                                 add=True)

        def wait_s(slot):
            for j in range(4):
                pltpu.make_async_copy(vals.at[slot, pl.ds(j * 128, 128)],
                                      acc_sh.at[pl.ds(0, 128)],
                                      sems[slot]).wait()

        for slot in (0, 1):
            load_idx_sync(slot, slot)
            fire_g(slot)
            wait_g(slot)
            fire_s(slot)

        @pl.loop(0, (UPT - 2) // 2)
        def _(p):
            u = 2 + p * 2
            for slot in (0, 1):
                wait_s(slot)
                load_idx_sync(slot, u + slot)
                fire_g(slot)
                wait_g(slot)
                fire_s(slot)

        wait_s(0)
        wait_s(1)

        plsc.subcore_barrier()
        last = (NTILE - 1) * RPT

        @pl.when(s < NTILE - 1)
        def _():
            pltpu.sync_copy(acc_sh.at[pl.ds(s * RPT, RPT)],
                            out_hbm.at[pl.ds(c * N + s * RPT, RPT)])

        @pl.when(s == NTILE - 1)
        def _():
            pltpu.sync_copy(acc_sh.at[pl.ds(last, N - last)],
                            out_hbm.at[pl.ds(c * N + last, N - last)])

    return k2(x, src3, dst2d)


U4 = 2048
UPT4 = EPAD // (NTILE * U4)


def _k4_seg1(y, src3, dst2d):
    @functools.partial(
        pl.kernel,
        mesh=_vmesh(),
        compiler_params=pltpu.CompilerParams(use_tc_tiling_on_sc=False),
        out_type=jax.ShapeDtypeStruct((B * N,), jnp.float32),
        scratch_types=[
            pltpu.VMEM((2, 16, 128), jnp.int32),
            pltpu.VMEM((2, 16, 128), jnp.int32),
            pltpu.VMEM((2, U4), jnp.float32),
            pltpu.VMEM((RPT,), jnp.float32),
            pltpu.VMEM_SHARED((NP,), jnp.float32),
            pltpu.SemaphoreType.DMA,
            pltpu.SemaphoreType.DMA,
            pltpu.SemaphoreType.DMA,
            pltpu.SemaphoreType.DMA,
            pltpu.SemaphoreType.DMA,
        ],
    )
    def k4(y_hbm, src_hbm, dst_hbm, out_hbm, sidx, didx, vals, zbuf, acc_sh,
           semi, semg0, semg1, sems0, sems1):
        c = lax.axis_index("c")
        s = lax.axis_index("s")
        semg = (semg0, semg1)
        sems = (sems0, sems1)

        _zero_vmem_1d(zbuf, RPT)
        pltpu.sync_copy(zbuf, acc_sh.at[pl.ds(s * RPT, RPT)])
        plsc.subcore_barrier()

        row0 = s * (UPT4 * 16)

        def load_idx_sync(slot, u):
            r = row0 + u * 16
            h1 = pltpu.async_copy(src_hbm.at[c, pl.ds(r, 16)], sidx.at[slot],
                                  semi)
            h2 = pltpu.async_copy(dst_hbm.at[pl.ds(r, 16)], didx.at[slot],
                                  semi)
            h1.wait()
            h2.wait()

        def fire_g(slot):
            for j in range(16):
                pltpu.async_copy(y_hbm.at[sidx.at[slot, j]],
                                 vals.at[slot, pl.ds(j * 128, 128)],
                                 semg[slot])

        def wait_g(slot):
            for j in range(16):
                pltpu.make_async_copy(y_hbm.at[pl.ds(0, 128)],
                                      vals.at[slot, pl.ds(j * 128, 128)],
                                      semg[slot]).wait()

        def fire_s(slot):
            for j in range(16):
                pltpu.async_copy(vals.at[slot, pl.ds(j * 128, 128)],
                                 acc_sh.at[didx.at[slot, j]], sems[slot],
                                 add=True)

        def wait_s(slot):
            for j in range(16):
                pltpu.make_async_copy(vals.at[slot, pl.ds(j * 128, 128)],
                                      acc_sh.at[pl.ds(0, 128)],
                                      sems[slot]).wait()

        for slot in (0, 1):
            load_idx_sync(slot, slot)
            fire_g(slot)
            wait_g(slot)
            fire_s(slot)

        @pl.loop(0, (UPT4 - 2) // 2)
        def _(p):
            u = 2 + p * 2
            for slot in (0, 1):
                wait_s(slot)
                load_idx_sync(slot, u + slot)
                fire_g(slot)
                wait_g(slot)
                fire_s(slot)

        wait_s(0)
        wait_s(1)

        plsc.subcore_barrier()
        last = (NTILE - 1) * RPT

        @pl.when(s < NTILE - 1)
        def _():
            pltpu.sync_copy(acc_sh.at[pl.ds(s * RPT, RPT)],
                            out_hbm.at[pl.ds(c * N + s * RPT, RPT)])

        @pl.when(s == NTILE - 1)
        def _():
            pltpu.sync_copy(acc_sh.at[pl.ds(last, N - last)],
                            out_hbm.at[pl.ds(c * N + last, N - last)])

    return k4(y, src3, dst2d)


def _k1_body(dyn_ref, sta_ref, wih_ref, whh_ref, bih_ref, bhh_ref, ws_ref,
             bs_ref, wf_ref, bf_ref, out_ref, h_ref, c_ref):
    t = pl.program_id(1)
    blk = out_ref.shape[0]
    dn = (((1,), (1,)), ((), ()))

    @pl.when(t == 0)
    def _():
        h_ref[...] = jnp.zeros((blk, H), jnp.float32)
        c_ref[...] = jnp.zeros((blk, H), jnp.float32)

    wcat = jnp.concatenate([wih_ref[...], whh_ref[...]], axis=1)
    bias = bih_ref[...] + bhh_ref[...]
    x_t = dyn_ref[0, 0]
    xh = jnp.concatenate([x_t, h_ref[...]], axis=1)
    gates = lax.dot_general(xh, wcat, dn,
                            preferred_element_type=jnp.float32) + bias
    gi = jax.nn.sigmoid(gates[:, 0 * H:1 * H])
    gf = jax.nn.sigmoid(gates[:, 1 * H:2 * H])
    gg = jnp.tanh(gates[:, 2 * H:3 * H])
    go = jax.nn.sigmoid(gates[:, 3 * H:4 * H])
    c = gf * c_ref[...] + gi * gg
    h = go * jnp.tanh(c)
    c_ref[...] = c
    h_ref[...] = h

    @pl.when(t == T - 1)
    def _():
        s_t = jax.nn.relu(
            lax.dot_general(sta_ref[0], ws_ref[...], dn,
                            preferred_element_type=jnp.float32) + bs_ref[...])
        wf = wf_ref[...]
        emb = jax.nn.relu(
            lax.dot_general(h, wf[:, :H], dn,
                            preferred_element_type=jnp.float32)
            + lax.dot_general(s_t, wf[:, H:], dn,
                              preferred_element_type=jnp.float32)
            + bf_ref[...])
        out_ref[...] = emb


def _k1_encode(dyn, sta, W_ih, W_hh, b_ih, b_hh, W_s, b_s, W_f, b_f):
    blk = 2000
    nb = N // blk
    grid = (B * nb, T)
    return pl.pallas_call(
        _k1_body,
        grid=grid,
        in_specs=[
            pl.BlockSpec((1, 1, blk, F_DYN), lambda i, t: (i // nb, t, i % nb, 0)),
            pl.BlockSpec((1, blk, F_STA), lambda i, t: (i // nb, i % nb, 0)),
            pl.BlockSpec((4 * H, F_DYN), lambda i, t: (0, 0)),
            pl.BlockSpec((4 * H, H), lambda i, t: (0, 0)),
            pl.BlockSpec((1, 4 * H), lambda i, t: (0, 0)),
            pl.BlockSpec((1, 4 * H), lambda i, t: (0, 0)),
            pl.BlockSpec((H, F_STA), lambda i, t: (0, 0)),
            pl.BlockSpec((1, H), lambda i, t: (0, 0)),
            pl.BlockSpec((H, 2 * H), lambda i, t: (0, 0)),
            pl.BlockSpec((1, H), lambda i, t: (0, 0)),
        ],
        out_specs=pl.BlockSpec((blk, H), lambda i, t: (i, 0)),
        out_shape=jax.ShapeDtypeStruct((B * N, H), jnp.float32),
        scratch_shapes=[
            pltpu.VMEM((blk, H), jnp.float32),
            pltpu.VMEM((blk, H), jnp.float32),
        ],
    )(dyn, sta, W_ih, W_hh, b_ih, b_hh, W_s, b_s, W_f, b_f)


def _k3_body(sum1_ref, x_ref, cnt_ref, wl1_ref, bl1_ref, wr1_ref, wl2_ref,
             wr2_ref, wo_ref, y_ref, z_ref):
    inv = 1.0 / jnp.maximum(cnt_ref[...], 1.0)
    mean = sum1_ref[:, :G] * inv
    dn = (((1,), (1,)), ((), ()))
    x1 = jax.nn.relu(
        lax.dot_general(mean, wl1_ref[...], dn, preferred_element_type=jnp.float32)
        + lax.dot_general(x_ref[:, :G], wr1_ref[...], dn,
                          preferred_element_type=jnp.float32)
        + bl1_ref[...])
    wo = wo_ref[...]
    vl = lax.dot_general(wo, wl2_ref[...], (((1,), (0,)), ((), ())),
                         preferred_element_type=jnp.float32)
    vr = lax.dot_general(wo, wr2_ref[...], (((1,), (0,)), ((), ())),
                         preferred_element_type=jnp.float32)
    y_ref[...] = jnp.sum(x1 * vl, axis=1, keepdims=True)
    z_ref[...] = jnp.sum(x1 * vr, axis=1, keepdims=True)


def _k3_sage1(sum1, embed, cnt_col, W_l1, b_l1, W_r1, W_l2, W_r2, W_o):
    blk = 2000
    grid = (B * N // blk,)
    return pl.pallas_call(
        _k3_body,
        grid=grid,
        in_specs=[
            pl.BlockSpec((blk, GP), lambda i: (i, 0)),
            pl.BlockSpec((blk, GP), lambda i: (i, 0)),
            pl.BlockSpec((blk, 1), lambda i: (i, 0)),
            pl.BlockSpec((G, G), lambda i: (0, 0)),
            pl.BlockSpec((1, G), lambda i: (0, 0)),
            pl.BlockSpec((G, G), lambda i: (0, 0)),
            pl.BlockSpec((G, G), lambda i: (0, 0)),
            pl.BlockSpec((G, G), lambda i: (0, 0)),
            pl.BlockSpec((1, G), lambda i: (0, 0)),
        ],
        out_specs=[
            pl.BlockSpec((blk, 1), lambda i: (i, 0)),
            pl.BlockSpec((blk, 1), lambda i: (i, 0)),
        ],
        out_shape=[
            jax.ShapeDtypeStruct((B * N, 1), jnp.float32),
            jax.ShapeDtypeStruct((B * N, 1), jnp.float32),
        ],
    )(sum1, embed, cnt_col, W_l1, b_l1, W_r1, W_l2, W_r2, W_o)


def _k5_body(s2_ref, cnt_ref, z_ref, bl2_ref, wo_ref, bo_ref, out_ref):
    inv = 1.0 / jnp.maximum(cnt_ref[0, :], 1.0)
    c0 = jnp.sum(bl2_ref[...] * wo_ref[...]) + bo_ref[0, 0]
    out_ref[...] = s2_ref[...] * inv[None, :] + z_ref[...] + c0


def _k5_combine(s2, cnt, z, b_l2, W_o, b_o):
    return pl.pallas_call(
        _k5_body,
        grid=(1,),
        in_specs=[
            pl.BlockSpec((B, N), lambda i: (0, 0)),
            pl.BlockSpec((1, N), lambda i: (0, 0)),
            pl.BlockSpec((B, N), lambda i: (0, 0)),
            pl.BlockSpec((1, G), lambda i: (0, 0)),
            pl.BlockSpec((1, G), lambda i: (0, 0)),
            pl.BlockSpec((1, 1), lambda i: (0, 0)),
        ],
        out_specs=pl.BlockSpec((B, N), lambda i: (0, 0)),
        out_shape=jax.ShapeDtypeStruct((B, N), jnp.float32),
    )(s2, cnt, z, b_l2, W_o, b_o)


def kernel(dynamic_features, static_features, edge_index, W_ih, W_hh, b_ih,
           b_hh, W_s, b_s, W_f, b_f, W_l1, b_l1, W_r1, W_l2, b_l2, W_r2, W_o,
           b_o):
    src = edge_index[0]
    dst = edge_index[1]
    npad = EPAD - E
    pad_src = (jnp.arange(npad, dtype=jnp.int32) * 37) % N
    pad_dst = N + (jnp.arange(npad, dtype=jnp.int32) % (NP - N))
    src2d = jnp.concatenate([src, pad_src]).reshape(EPAD // 128, 128)
    dst2d = jnp.concatenate([dst, pad_dst]).reshape(EPAD // 128, 128)
    src3 = jnp.stack([src2d, src2d + N])

    cnt = _k0_counts(dst2d)
    cnt_n = cnt[:N]
    cnt_col = jnp.concatenate([cnt_n, cnt_n]).reshape(B * N, 1)

    embed = _k1_encode(dynamic_features, static_features, W_ih, W_hh,
                       b_ih.reshape(1, 4 * H), b_hh.reshape(1, 4 * H), W_s,
                       b_s.reshape(1, H), W_f, b_f.reshape(1, H))

    sum1 = _k2_seg64(embed, src3, dst2d)

    y1, z1 = _k3_sage1(sum1, embed, cnt_col, W_l1, b_l1.reshape(1, G), W_r1,
                       W_l2, W_r2, W_o)

    s2 = _k4_seg1(y1.reshape(B * N), src3, dst2d).reshape(B, N)

    z = z1.reshape(B, N)
    pred = _k5_combine(s2, cnt_n.reshape(1, N), z, b_l2.reshape(1, G), W_o,
                       b_o.reshape(1, 1))
    return pred


# speedup vs baseline: 1.0305x; 1.0305x over previous
import functools

import jax
import jax.numpy as jnp
from jax import lax
from jax.experimental import pallas as pl
from jax.experimental.pallas import tpu as pltpu
from jax.experimental.pallas import tpu_sc as plsc

B, T, N, F_DYN, F_STA = 2, 12, 10000, 16, 16
H, G, E = 64, 64, 320000

NTILE = 16
NCORE = 2
EPAD = 327680
UNIT = 512
UPT = EPAD // (NTILE * UNIT)
NP = 10240
RPT = NP // NTILE
GP = 64
FW = T * F_DYN + F_STA


def _vmesh():
    return plsc.VectorSubcoreMesh(core_axis_name="c", subcore_axis_name="s")


def _zero_vmem_2d(buf, rows, cols):
    zv = jnp.zeros((16,), jnp.float32)

    @pl.loop(0, rows)
    def _(r):
        @pl.loop(0, cols // 16)
        def _(j):
            buf[r, pl.ds(j * 16, 16)] = zv


def _zero_vmem_1d(buf, n):
    zv = jnp.zeros((16,), jnp.float32)

    @pl.loop(0, n // 16)
    def _(j):
        buf[pl.ds(j * 16, 16)] = zv


def _k0_counts(dst2d):
    @functools.partial(
        pl.kernel,
        mesh=_vmesh(),
        compiler_params=pltpu.CompilerParams(use_tc_tiling_on_sc=False),
        out_type=jax.ShapeDtypeStruct((NP,), jnp.float32),
        scratch_types=[
            pltpu.VMEM((4, 128), jnp.int32),
            pltpu.VMEM((128,), jnp.float32),
            pltpu.VMEM((RPT,), jnp.float32),
            pltpu.VMEM_SHARED((NP,), jnp.float32),
        ],
    )
    def k0(dst_hbm, out_hbm, didx, ones_v, zbuf, cnt_sh):
        c = lax.axis_index("c")
        s = lax.axis_index("s")

        @pl.when(c == 0)
        def _():
            ov = jnp.ones((16,), jnp.float32)

            @pl.loop(0, 8)
            def _(j):
                ones_v[pl.ds(j * 16, 16)] = ov

            _zero_vmem_1d(zbuf, RPT)
            pltpu.sync_copy(zbuf, cnt_sh.at[pl.ds(s * RPT, RPT)])
            plsc.subcore_barrier()

            row0 = s * (UPT * 4)

            @pl.loop(0, UPT)
            def _(u):
                pltpu.sync_copy(dst_hbm.at[pl.ds(row0 + u * 4, 4)], didx)
                for j in range(4):
                    pltpu.sync_copy(ones_v, cnt_sh.at[didx.at[j]], add=True)

            plsc.subcore_barrier()
            pltpu.sync_copy(cnt_sh.at[pl.ds(s * RPT, RPT)],
                            out_hbm.at[pl.ds(s * RPT, RPT)])

    return k0(dst2d)


def _k2_seg64(x, src3, dst2d):
    @functools.partial(
        pl.kernel,
        mesh=_vmesh(),
        compiler_params=pltpu.CompilerParams(use_tc_tiling_on_sc=False),
        out_type=jax.ShapeDtypeStruct((B * N, GP), jnp.float32),
        scratch_types=[
            pltpu.VMEM((2, 4, 128), jnp.int32),
            pltpu.VMEM((2, 4, 128), jnp.int32),
            pltpu.VMEM((2, UNIT, GP), jnp.float32),
            pltpu.VMEM((128, GP), jnp.float32),
            pltpu.VMEM_SHARED((NP, GP), jnp.float32),
            pltpu.SemaphoreType.DMA,
            pltpu.SemaphoreType.DMA,
            pltpu.SemaphoreType.DMA,
            pltpu.SemaphoreType.DMA,
            pltpu.SemaphoreType.DMA,
        ],
    )
    def k2(x_hbm, src_hbm, dst_hbm, out_hbm, sidx, didx, vals, zbuf, acc_sh,
           semi, semg0, semg1, sems0, sems1):
        c = lax.axis_index("c")
        s = lax.axis_index("s")
        semg = (semg0, semg1)
        sems = (sems0, sems1)

        _zero_vmem_2d(zbuf, 128, GP)

        @pl.loop(0, RPT // 128)
        def _(k):
            pltpu.sync_copy(zbuf, acc_sh.at[pl.ds(s * RPT + k * 128, 128)])

        plsc.subcore_barrier()

        row0 = s * (UPT * 4)

        def load_idx_sync(slot, u):
            r = row0 + u * 4
            h1 = pltpu.async_copy(src_hbm.at[c, pl.ds(r, 4)], sidx.at[slot],
                                  semi)
            h2 = pltpu.async_copy(dst_hbm.at[pl.ds(r, 4)], didx.at[slot], semi)
            h1.wait()
            h2.wait()

        def fire_g(slot):
            for j in range(4):
                pltpu.async_copy(x_hbm.at[sidx.at[slot, j]],
                                 vals.at[slot, pl.ds(j * 128, 128)],
                                 semg[slot])

        def wait_g(slot):
            for j in range(4):
                pltpu.make_async_copy(x_hbm.at[pl.ds(0, 128)],
                                      vals.at[slot, pl.ds(j * 128, 128)],
                                      semg[slot]).wait()

        def fire_s(slot):
            for j in range(4):
                pltpu.async_copy(vals.at[slot, pl.ds(j * 128, 128)],
                                 acc_sh.at[didx.at[slot, j]], sems[slot],
                                 add=True)

        def wait_s(slot):
            for j in range(4):
                pltpu.make_async_copy(vals.at[slot, pl.ds(j * 128, 128)],
                                      acc_sh.at[pl.ds(0, 128)],
                                      sems[slot]).wait()

        for slot in (0, 1):
            load_idx_sync(slot, slot)
            fire_g(slot)
            wait_g(slot)
            fire_s(slot)

        @pl.loop(0, (UPT - 2) // 2)
        def _(p):
            u = 2 + p * 2
            for slot in (0, 1):
                wait_s(slot)
                load_idx_sync(slot, u + slot)
                fire_g(slot)
                wait_g(slot)
                fire_s(slot)

        wait_s(0)
        wait_s(1)

        plsc.subcore_barrier()
        last = (NTILE - 1) * RPT

        @pl.when(s < NTILE - 1)
        def _():
            pltpu.sync_copy(acc_sh.at[pl.ds(s * RPT, RPT)],
                            out_hbm.at[pl.ds(c * N + s * RPT, RPT)])

        @pl.when(s == NTILE - 1)
        def _():
            pltpu.sync_copy(acc_sh.at[pl.ds(last, N - last)],
                            out_hbm.at[pl.ds(c * N + last, N - last)])

    return k2(x, src3, dst2d)


U4 = 2048
UPT4 = EPAD // (NTILE * U4)


def _k4_seg1(y, src3, dst2d):
    @functools.partial(
        pl.kernel,
        mesh=_vmesh(),
        compiler_params=pltpu.CompilerParams(use_tc_tiling_on_sc=False),
        out_type=jax.ShapeDtypeStruct((B * N,), jnp.float32),
        scratch_types=[
            pltpu.VMEM((2, 16, 128), jnp.int32),
            pltpu.VMEM((2, 16, 128), jnp.int32),
            pltpu.VMEM((2, U4), jnp.float32),
            pltpu.VMEM((RPT,), jnp.float32),
            pltpu.VMEM_SHARED((NP,), jnp.float32),
            pltpu.SemaphoreType.DMA,
            pltpu.SemaphoreType.DMA,
            pltpu.SemaphoreType.DMA,
            pltpu.SemaphoreType.DMA,
            pltpu.SemaphoreType.DMA,
        ],
    )
    def k4(y_hbm, src_hbm, dst_hbm, out_hbm, sidx, didx, vals, zbuf, acc_sh,
           semi, semg0, semg1, sems0, sems1):
        c = lax.axis_index("c")
        s = lax.axis_index("s")
        semg = (semg0, semg1)
        sems = (sems0, sems1)

        _zero_vmem_1d(zbuf, RPT)
        pltpu.sync_copy(zbuf, acc_sh.at[pl.ds(s * RPT, RPT)])
        plsc.subcore_barrier()

        row0 = s * (UPT4 * 16)

        def load_idx_sync(slot, u):
            r = row0 + u * 16
            h1 = pltpu.async_copy(src_hbm.at[c, pl.ds(r, 16)], sidx.at[slot],
                                  semi)
            h2 = pltpu.async_copy(dst_hbm.at[pl.ds(r, 16)], didx.at[slot],
                                  semi)
            h1.wait()
            h2.wait()

        def fire_g(slot):
            for j in range(16):
                pltpu.async_copy(y_hbm.at[sidx.at[slot, j]],
                                 vals.at[slot, pl.ds(j * 128, 128)],
                                 semg[slot])

        def wait_g(slot):
            for j in range(16):
                pltpu.make_async_copy(y_hbm.at[pl.ds(0, 128)],
                                      vals.at[slot, pl.ds(j * 128, 128)],
                                      semg[slot]).wait()

        def fire_s(slot):
            for j in range(16):
                pltpu.async_copy(vals.at[slot, pl.ds(j * 128, 128)],
                                 acc_sh.at[didx.at[slot, j]], sems[slot],
                                 add=True)

        def wait_s(slot):
            for j in range(16):
                pltpu.make_async_copy(vals.at[slot, pl.ds(j * 128, 128)],
                                      acc_sh.at[pl.ds(0, 128)],
                                      sems[slot]).wait()

        for slot in (0, 1):
            load_idx_sync(slot, slot)
            fire_g(slot)
            wait_g(slot)
            fire_s(slot)

        @pl.loop(0, (UPT4 - 2) // 2)
        def _(p):
            u = 2 + p * 2
            for slot in (0, 1):
                wait_s(slot)
                load_idx_sync(slot, u + slot)
                fire_g(slot)
                wait_g(slot)
                fire_s(slot)

        wait_s(0)
        wait_s(1)

        plsc.subcore_barrier()
        last = (NTILE - 1) * RPT

        @pl.when(s < NTILE - 1)
        def _():
            pltpu.sync_copy(acc_sh.at[pl.ds(s * RPT, RPT)],
                            out_hbm.at[pl.ds(c * N + s * RPT, RPT)])

        @pl.when(s == NTILE - 1)
        def _():
            pltpu.sync_copy(acc_sh.at[pl.ds(last, N - last)],
                            out_hbm.at[pl.ds(c * N + last, N - last)])

    return k4(y, src3, dst2d)


def _k1_body(dyn_ref, sta_ref, wih_ref, whh_ref, bih_ref, bhh_ref, ws_ref,
             bs_ref, wf_ref, bf_ref, out_ref, h_ref, c_ref):
    t = pl.program_id(1)
    blk = out_ref.shape[0]
    dn = (((1,), (1,)), ((), ()))

    @pl.when(t == 0)
    def _():
        h_ref[...] = jnp.zeros((blk, H), jnp.float32)
        c_ref[...] = jnp.zeros((blk, H), jnp.float32)

    bias = bih_ref[...] + bhh_ref[...]
    x_t = dyn_ref[0, 0]
    gates = (lax.dot_general(x_t, wih_ref[...], dn,
                             preferred_element_type=jnp.float32)
             + lax.dot_general(h_ref[...], whh_ref[...], dn,
                               preferred_element_type=jnp.float32)
             + bias)
    gi = jax.nn.sigmoid(gates[:, 0 * H:1 * H])
    gf = jax.nn.sigmoid(gates[:, 1 * H:2 * H])
    gg = jnp.tanh(gates[:, 2 * H:3 * H])
    go = jax.nn.sigmoid(gates[:, 3 * H:4 * H])
    c = gf * c_ref[...] + gi * gg
    h = go * jnp.tanh(c)
    c_ref[...] = c
    h_ref[...] = h

    @pl.when(t == T - 1)
    def _():
        s_t = jax.nn.relu(
            lax.dot_general(sta_ref[0], ws_ref[...], dn,
                            preferred_element_type=jnp.float32) + bs_ref[...])
        wf = wf_ref[...]
        emb = jax.nn.relu(
            lax.dot_general(h, wf[:, :H], dn,
                            preferred_element_type=jnp.float32)
            + lax.dot_general(s_t, wf[:, H:], dn,
                              preferred_element_type=jnp.float32)
            + bf_ref[...])
        out_ref[...] = emb


def _k1_encode(dyn, sta, W_ih, W_hh, b_ih, b_hh, W_s, b_s, W_f, b_f):
    blk = 2000
    nb = N // blk
    grid = (B * nb, T)
    return pl.pallas_call(
        _k1_body,
        grid=grid,
        in_specs=[
            pl.BlockSpec((1, 1, blk, F_DYN), lambda i, t: (i // nb, t, i % nb, 0)),
            pl.BlockSpec((1, blk, F_STA), lambda i, t: (i // nb, i % nb, 0)),
            pl.BlockSpec((4 * H, F_DYN), lambda i, t: (0, 0)),
            pl.BlockSpec((4 * H, H), lambda i, t: (0, 0)),
            pl.BlockSpec((1, 4 * H), lambda i, t: (0, 0)),
            pl.BlockSpec((1, 4 * H), lambda i, t: (0, 0)),
            pl.BlockSpec((H, F_STA), lambda i, t: (0, 0)),
            pl.BlockSpec((1, H), lambda i, t: (0, 0)),
            pl.BlockSpec((H, 2 * H), lambda i, t: (0, 0)),
            pl.BlockSpec((1, H), lambda i, t: (0, 0)),
        ],
        out_specs=pl.BlockSpec((blk, H), lambda i, t: (i, 0)),
        out_shape=jax.ShapeDtypeStruct((B * N, H), jnp.float32),
        scratch_shapes=[
            pltpu.VMEM((blk, H), jnp.float32),
            pltpu.VMEM((blk, H), jnp.float32),
        ],
    )(dyn, sta, W_ih, W_hh, b_ih, b_hh, W_s, b_s, W_f, b_f)


def _k3_body(sum1_ref, x_ref, cnt_ref, wl1_ref, bl1_ref, wr1_ref, wl2_ref,
             wr2_ref, wo_ref, y_ref, z_ref):
    inv = 1.0 / jnp.maximum(cnt_ref[...], 1.0)
    mean = sum1_ref[:, :G] * inv
    dn = (((1,), (1,)), ((), ()))
    x1 = jax.nn.relu(
        lax.dot_general(mean, wl1_ref[...], dn, preferred_element_type=jnp.float32)
        + lax.dot_general(x_ref[:, :G], wr1_ref[...], dn,
                          preferred_element_type=jnp.float32)
        + bl1_ref[...])
    wo = wo_ref[...]
    vl = lax.dot_general(wo, wl2_ref[...], (((1,), (0,)), ((), ())),
                         preferred_element_type=jnp.float32)
    vr = lax.dot_general(wo, wr2_ref[...], (((1,), (0,)), ((), ())),
                         preferred_element_type=jnp.float32)
    y_ref[...] = jnp.sum(x1 * vl, axis=1, keepdims=True)
    z_ref[...] = jnp.sum(x1 * vr, axis=1, keepdims=True)


def _k3_sage1(sum1, embed, cnt_col, W_l1, b_l1, W_r1, W_l2, W_r2, W_o):
    blk = 2000
    grid = (B * N // blk,)
    return pl.pallas_call(
        _k3_body,
        grid=grid,
        in_specs=[
            pl.BlockSpec((blk, GP), lambda i: (i, 0)),
            pl.BlockSpec((blk, GP), lambda i: (i, 0)),
            pl.BlockSpec((blk, 1), lambda i: (i, 0)),
            pl.BlockSpec((G, G), lambda i: (0, 0)),
            pl.BlockSpec((1, G), lambda i: (0, 0)),
            pl.BlockSpec((G, G), lambda i: (0, 0)),
            pl.BlockSpec((G, G), lambda i: (0, 0)),
            pl.BlockSpec((G, G), lambda i: (0, 0)),
            pl.BlockSpec((1, G), lambda i: (0, 0)),
        ],
        out_specs=[
            pl.BlockSpec((blk, 1), lambda i: (i, 0)),
            pl.BlockSpec((blk, 1), lambda i: (i, 0)),
        ],
        out_shape=[
            jax.ShapeDtypeStruct((B * N, 1), jnp.float32),
            jax.ShapeDtypeStruct((B * N, 1), jnp.float32),
        ],
    )(sum1, embed, cnt_col, W_l1, b_l1, W_r1, W_l2, W_r2, W_o)


def _k5_body(s2_ref, cnt_ref, z_ref, bl2_ref, wo_ref, bo_ref, out_ref):
    inv = 1.0 / jnp.maximum(cnt_ref[0, :], 1.0)
    c0 = jnp.sum(bl2_ref[...] * wo_ref[...]) + bo_ref[0, 0]
    out_ref[...] = s2_ref[...] * inv[None, :] + z_ref[...] + c0


def _k5_combine(s2, cnt, z, b_l2, W_o, b_o):
    return pl.pallas_call(
        _k5_body,
        grid=(1,),
        in_specs=[
            pl.BlockSpec((B, N), lambda i: (0, 0)),
            pl.BlockSpec((1, N), lambda i: (0, 0)),
            pl.BlockSpec((B, N), lambda i: (0, 0)),
            pl.BlockSpec((1, G), lambda i: (0, 0)),
            pl.BlockSpec((1, G), lambda i: (0, 0)),
            pl.BlockSpec((1, 1), lambda i: (0, 0)),
        ],
        out_specs=pl.BlockSpec((B, N), lambda i: (0, 0)),
        out_shape=jax.ShapeDtypeStruct((B, N), jnp.float32),
    )(s2, cnt, z, b_l2, W_o, b_o)


def kernel(dynamic_features, static_features, edge_index, W_ih, W_hh, b_ih,
           b_hh, W_s, b_s, W_f, b_f, W_l1, b_l1, W_r1, W_l2, b_l2, W_r2, W_o,
           b_o):
    src = edge_index[0]
    dst = edge_index[1]
    npad = EPAD - E
    pad_src = (jnp.arange(npad, dtype=jnp.int32) * 37) % N
    pad_dst = N + (jnp.arange(npad, dtype=jnp.int32) % (NP - N))
    src2d = jnp.concatenate([src, pad_src]).reshape(EPAD // 128, 128)
    dst2d = jnp.concatenate([dst, pad_dst]).reshape(EPAD // 128, 128)
    src3 = jnp.stack([src2d, src2d + N])

    cnt = _k0_counts(dst2d)
    cnt_n = cnt[:N]
    cnt_col = jnp.concatenate([cnt_n, cnt_n]).reshape(B * N, 1)

    embed = _k1_encode(dynamic_features, static_features, W_ih, W_hh,
                       b_ih.reshape(1, 4 * H), b_hh.reshape(1, 4 * H), W_s,
                       b_s.reshape(1, H), W_f, b_f.reshape(1, H))

    sum1 = _k2_seg64(embed, src3, dst2d)

    y1, z1 = _k3_sage1(sum1, embed, cnt_col, W_l1, b_l1.reshape(1, G), W_r1,
                       W_l2, W_r2, W_o)

    s2 = _k4_seg1(y1.reshape(B * N), src3, dst2d).reshape(B, N)

    z = z1.reshape(B, N)
    pred = _k5_combine(s2, cnt_n.reshape(1, N), z, b_l2.reshape(1, G), W_o,
                       b_o.reshape(1, 1))
    return pred


# speedup vs baseline: 1.2023x; 1.1668x over previous
import functools

import jax
import jax.numpy as jnp
from jax import lax
from jax.experimental import pallas as pl
from jax.experimental.pallas import tpu as pltpu
from jax.experimental.pallas import tpu_sc as plsc

B, T, N, F_DYN, F_STA = 2, 12, 10000, 16, 16
H, G, E = 64, 64, 320000

NTILE = 16
NCORE = 2
EPAD = 327680
UNIT = 512
UPT = EPAD // (NTILE * UNIT)
NP = 10240
RPT = NP // NTILE
GP = 64
FW = T * F_DYN + F_STA


def _vmesh():
    return plsc.VectorSubcoreMesh(core_axis_name="c", subcore_axis_name="s")


def _zero_vmem_2d(buf, rows, cols):
    zv = jnp.zeros((16,), jnp.float32)

    @pl.loop(0, rows)
    def _(r):
        @pl.loop(0, cols // 16)
        def _(j):
            buf[r, pl.ds(j * 16, 16)] = zv


def _zero_vmem_1d(buf, n):
    zv = jnp.zeros((16,), jnp.float32)

    @pl.loop(0, n // 16)
    def _(j):
        buf[pl.ds(j * 16, 16)] = zv


def _k0_counts(dst2d):
    @functools.partial(
        pl.kernel,
        mesh=_vmesh(),
        compiler_params=pltpu.CompilerParams(use_tc_tiling_on_sc=False),
        out_type=jax.ShapeDtypeStruct((NP,), jnp.float32),
        scratch_types=[
            pltpu.VMEM((4, 128), jnp.int32),
            pltpu.VMEM((128,), jnp.float32),
            pltpu.VMEM((RPT,), jnp.float32),
            pltpu.VMEM_SHARED((NP,), jnp.float32),
        ],
    )
    def k0(dst_hbm, out_hbm, didx, ones_v, zbuf, cnt_sh):
        c = lax.axis_index("c")
        s = lax.axis_index("s")

        @pl.when(c == 0)
        def _():
            ov = jnp.ones((16,), jnp.float32)

            @pl.loop(0, 8)
            def _(j):
                ones_v[pl.ds(j * 16, 16)] = ov

            _zero_vmem_1d(zbuf, RPT)
            pltpu.sync_copy(zbuf, cnt_sh.at[pl.ds(s * RPT, RPT)])
            plsc.subcore_barrier()

            row0 = s * (UPT * 4)

            @pl.loop(0, UPT)
            def _(u):
                pltpu.sync_copy(dst_hbm.at[pl.ds(row0 + u * 4, 4)], didx)
                for j in range(4):
                    pltpu.sync_copy(ones_v, cnt_sh.at[didx.at[j]], add=True)

            plsc.subcore_barrier()
            pltpu.sync_copy(cnt_sh.at[pl.ds(s * RPT, RPT)],
                            out_hbm.at[pl.ds(s * RPT, RPT)])

    return k0(dst2d)


def _k2_seg64(x, src3, dst2d):
    @functools.partial(
        pl.kernel,
        mesh=_vmesh(),
        compiler_params=pltpu.CompilerParams(use_tc_tiling_on_sc=False),
        out_type=jax.ShapeDtypeStruct((B * N, GP), jnp.float32),
        scratch_types=[
            pltpu.VMEM((2, 4, 128), jnp.int32),
            pltpu.VMEM((2, 4, 128), jnp.int32),
            pltpu.VMEM((2, UNIT, GP), jnp.float32),
            pltpu.VMEM((128, GP), jnp.float32),
            pltpu.VMEM_SHARED((NP, GP), jnp.float32),
            pltpu.SemaphoreType.DMA,
            pltpu.SemaphoreType.DMA,
            pltpu.SemaphoreType.DMA,
            pltpu.SemaphoreType.DMA,
            pltpu.SemaphoreType.DMA,
        ],
    )
    def k2(x_hbm, src_hbm, dst_hbm, out_hbm, sidx, didx, vals, zbuf, acc_sh,
           semi, semg0, semg1, sems0, sems1):
        c = lax.axis_index("c")
        s = lax.axis_index("s")
        semg = (semg0, semg1)
        sems = (sems0, sems1)

        _zero_vmem_2d(zbuf, 128, GP)

        @pl.loop(0, RPT // 128)
        def _(k):
            pltpu.sync_copy(zbuf, acc_sh.at[pl.ds(s * RPT + k * 128, 128)])

        plsc.subcore_barrier()

        row0 = s * (UPT * 4)

        def load_idx_sync(slot, u):
            r = row0 + u * 4
            h1 = pltpu.async_copy(src_hbm.at[c, pl.ds(r, 4)], sidx.at[slot],
                                  semi)
            h2 = pltpu.async_copy(dst_hbm.at[pl.ds(r, 4)], didx.at[slot], semi)
            h1.wait()
            h2.wait()

        def fire_g(slot):
            for j in range(4):
                pltpu.async_copy(x_hbm.at[sidx.at[slot, j]],
                                 vals.at[slot, pl.ds(j * 128, 128)],
                                 semg[slot])

        def wait_g(slot):
            for j in range(4):
                pltpu.make_async_copy(x_hbm.at[pl.ds(0, 128)],
                                      vals.at[slot, pl.ds(j * 128, 128)],
                                      semg[slot]).wait()

        def fire_s(slot):
            for j in range(4):
                pltpu.async_copy(vals.at[slot, pl.ds(j * 128, 128)],
                                 acc_sh.at[didx.at[slot, j]], sems[slot],
                                 add=True)

        def wait_s(slot):
            for j in range(4):
                pltpu.make_async_copy(vals.at[slot, pl.ds(j * 128, 128)],
                                      acc_sh.at[pl.ds(0, 128)],
                                      sems[slot]).wait()

        for slot in (0, 1):
            load_idx_sync(slot, slot)
            fire_g(slot)
            wait_g(slot)
            fire_s(slot)

        @pl.loop(0, (UPT - 2) // 2)
        def _(p):
            u = 2 + p * 2
            for slot in (0, 1):
                wait_s(slot)
                load_idx_sync(slot, u + slot)
                fire_g(slot)
                wait_g(slot)
                fire_s(slot)

        wait_s(0)
        wait_s(1)

        plsc.subcore_barrier()
        last = (NTILE - 1) * RPT

        @pl.when(s < NTILE - 1)
        def _():
            pltpu.sync_copy(acc_sh.at[pl.ds(s * RPT, RPT)],
                            out_hbm.at[pl.ds(c * N + s * RPT, RPT)])

        @pl.when(s == NTILE - 1)
        def _():
            pltpu.sync_copy(acc_sh.at[pl.ds(last, N - last)],
                            out_hbm.at[pl.ds(c * N + last, N - last)])

    return k2(x, src3, dst2d)


U4 = 2048
UPT4 = EPAD // (NTILE * U4)


def _k4_seg1(y, src3, dst2d):
    @functools.partial(
        pl.kernel,
        mesh=_vmesh(),
        compiler_params=pltpu.CompilerParams(use_tc_tiling_on_sc=False),
        out_type=jax.ShapeDtypeStruct((B * N,), jnp.float32),
        scratch_types=[
            pltpu.VMEM((2, 16, 128), jnp.int32),
            pltpu.VMEM((2, 16, 128), jnp.int32),
            pltpu.VMEM((2, U4), jnp.float32),
            pltpu.VMEM((RPT,), jnp.float32),
            pltpu.VMEM_SHARED((NP,), jnp.float32),
            pltpu.SemaphoreType.DMA,
            pltpu.SemaphoreType.DMA,
            pltpu.SemaphoreType.DMA,
            pltpu.SemaphoreType.DMA,
            pltpu.SemaphoreType.DMA,
        ],
    )
    def k4(y_hbm, src_hbm, dst_hbm, out_hbm, sidx, didx, vals, zbuf, acc_sh,
           semi, semg0, semg1, sems0, sems1):
        c = lax.axis_index("c")
        s = lax.axis_index("s")
        semg = (semg0, semg1)
        sems = (sems0, sems1)

        _zero_vmem_1d(zbuf, RPT)
        pltpu.sync_copy(zbuf, acc_sh.at[pl.ds(s * RPT, RPT)])
        plsc.subcore_barrier()

        row0 = s * (UPT4 * 16)

        def load_idx_sync(slot, u):
            r = row0 + u * 16
            h1 = pltpu.async_copy(src_hbm.at[c, pl.ds(r, 16)], sidx.at[slot],
                                  semi)
            h2 = pltpu.async_copy(dst_hbm.at[pl.ds(r, 16)], didx.at[slot],
                                  semi)
            h1.wait()
            h2.wait()

        def fire_g(slot):
            for j in range(16):
                pltpu.async_copy(y_hbm.at[sidx.at[slot, j]],
                                 vals.at[slot, pl.ds(j * 128, 128)],
                                 semg[slot])

        def wait_g(slot):
            for j in range(16):
                pltpu.make_async_copy(y_hbm.at[pl.ds(0, 128)],
                                      vals.at[slot, pl.ds(j * 128, 128)],
                                      semg[slot]).wait()

        def fire_s(slot):
            for j in range(16):
                pltpu.async_copy(vals.at[slot, pl.ds(j * 128, 128)],
                                 acc_sh.at[didx.at[slot, j]], sems[slot],
                                 add=True)

        def wait_s(slot):
            for j in range(16):
                pltpu.make_async_copy(vals.at[slot, pl.ds(j * 128, 128)],
                                      acc_sh.at[pl.ds(0, 128)],
                                      sems[slot]).wait()

        for slot in (0, 1):
            load_idx_sync(slot, slot)
            fire_g(slot)
            wait_g(slot)
            fire_s(slot)

        @pl.loop(0, (UPT4 - 2) // 2)
        def _(p):
            u = 2 + p * 2
            for slot in (0, 1):
                wait_s(slot)
                load_idx_sync(slot, u + slot)
                fire_g(slot)
                wait_g(slot)
                fire_s(slot)

        wait_s(0)
        wait_s(1)

        plsc.subcore_barrier()
        last = (NTILE - 1) * RPT

        @pl.when(s < NTILE - 1)
        def _():
            pltpu.sync_copy(acc_sh.at[pl.ds(s * RPT, RPT)],
                            out_hbm.at[pl.ds(c * N + s * RPT, RPT)])

        @pl.when(s == NTILE - 1)
        def _():
            pltpu.sync_copy(acc_sh.at[pl.ds(last, N - last)],
                            out_hbm.at[pl.ds(c * N + last, N - last)])

    return k4(y, src3, dst2d)


def _k1_body(*refs):
    dyn_refs = refs[:T]
    sta_ref, wih_ref, whh_ref, bih_ref, bhh_ref, ws_ref, bs_ref, wf_ref, \
        bf_ref, out_ref = refs[T:]
    blk = out_ref.shape[0]
    dn = (((1,), (1,)), ((), ()))
    wih = wih_ref[...]
    whh = whh_ref[...]
    bias = bih_ref[...] + bhh_ref[...]
    h = None
    c = None
    for t in range(T):
        x_t = dyn_refs[t][0, 0]
        gates = lax.dot_general(x_t, wih, dn,
                                preferred_element_type=jnp.float32) + bias
        if h is not None:
            gates = gates + lax.dot_general(h, whh, dn,
                                            preferred_element_type=jnp.float32)
        gi = jax.nn.sigmoid(gates[:, 0 * H:1 * H])
        gf = jax.nn.sigmoid(gates[:, 1 * H:2 * H])
        gg = jnp.tanh(gates[:, 2 * H:3 * H])
        go = jax.nn.sigmoid(gates[:, 3 * H:4 * H])
        c = gi * gg if c is None else gf * c + gi * gg
        h = go * jnp.tanh(c)
    s_t = jax.nn.relu(
        lax.dot_general(sta_ref[0], ws_ref[...], dn,
                        preferred_element_type=jnp.float32) + bs_ref[...])
    wf = wf_ref[...]
    emb = jax.nn.relu(
        lax.dot_general(h, wf[:, :H], dn, preferred_element_type=jnp.float32)
        + lax.dot_general(s_t, wf[:, H:], dn,
                          preferred_element_type=jnp.float32)
        + bf_ref[...])
    out_ref[...] = emb


def _k1_encode(dyn, sta, W_ih, W_hh, b_ih, b_hh, W_s, b_s, W_f, b_f):
    blk = 1000
    nb = N // blk
    grid = (B * nb,)
    dyn_specs = [
        pl.BlockSpec((1, 1, blk, F_DYN), lambda i, t=t: (i // nb, t, i % nb, 0))
        for t in range(T)
    ]
    return pl.pallas_call(
        _k1_body,
        grid=grid,
        in_specs=dyn_specs + [
            pl.BlockSpec((1, blk, F_STA), lambda i: (i // nb, i % nb, 0)),
            pl.BlockSpec((4 * H, F_DYN), lambda i: (0, 0)),
            pl.BlockSpec((4 * H, H), lambda i: (0, 0)),
            pl.BlockSpec((1, 4 * H), lambda i: (0, 0)),
            pl.BlockSpec((1, 4 * H), lambda i: (0, 0)),
            pl.BlockSpec((H, F_STA), lambda i: (0, 0)),
            pl.BlockSpec((1, H), lambda i: (0, 0)),
            pl.BlockSpec((H, 2 * H), lambda i: (0, 0)),
            pl.BlockSpec((1, H), lambda i: (0, 0)),
        ],
        out_specs=pl.BlockSpec((blk, H), lambda i: (i, 0)),
        out_shape=jax.ShapeDtypeStruct((B * N, H), jnp.float32),
    )(*([dyn] * T), sta, W_ih, W_hh, b_ih, b_hh, W_s, b_s, W_f, b_f)


def _k3_body(sum1_ref, x_ref, cnt_ref, wl1_ref, bl1_ref, wr1_ref, wl2_ref,
             wr2_ref, wo_ref, y_ref, z_ref):
    inv = 1.0 / jnp.maximum(cnt_ref[...], 1.0)
    mean = sum1_ref[:, :G] * inv
    dn = (((1,), (1,)), ((), ()))
    x1 = jax.nn.relu(
        lax.dot_general(mean, wl1_ref[...], dn, preferred_element_type=jnp.float32)
        + lax.dot_general(x_ref[:, :G], wr1_ref[...], dn,
                          preferred_element_type=jnp.float32)
        + bl1_ref[...])
    wo = wo_ref[...]
    vl = lax.dot_general(wo, wl2_ref[...], (((1,), (0,)), ((), ())),
                         preferred_element_type=jnp.float32)
    vr = lax.dot_general(wo, wr2_ref[...], (((1,), (0,)), ((), ())),
                         preferred_element_type=jnp.float32)
    y_ref[...] = jnp.sum(x1 * vl, axis=1, keepdims=True)
    z_ref[...] = jnp.sum(x1 * vr, axis=1, keepdims=True)


def _k3_sage1(sum1, embed, cnt_col, W_l1, b_l1, W_r1, W_l2, W_r2, W_o):
    blk = 2000
    grid = (B * N // blk,)
    return pl.pallas_call(
        _k3_body,
        grid=grid,
        in_specs=[
            pl.BlockSpec((blk, GP), lambda i: (i, 0)),
            pl.BlockSpec((blk, GP), lambda i: (i, 0)),
            pl.BlockSpec((blk, 1), lambda i: (i, 0)),
            pl.BlockSpec((G, G), lambda i: (0, 0)),
            pl.BlockSpec((1, G), lambda i: (0, 0)),
            pl.BlockSpec((G, G), lambda i: (0, 0)),
            pl.BlockSpec((G, G), lambda i: (0, 0)),
            pl.BlockSpec((G, G), lambda i: (0, 0)),
            pl.BlockSpec((1, G), lambda i: (0, 0)),
        ],
        out_specs=[
            pl.BlockSpec((blk, 1), lambda i: (i, 0)),
            pl.BlockSpec((blk, 1), lambda i: (i, 0)),
        ],
        out_shape=[
            jax.ShapeDtypeStruct((B * N, 1), jnp.float32),
            jax.ShapeDtypeStruct((B * N, 1), jnp.float32),
        ],
    )(sum1, embed, cnt_col, W_l1, b_l1, W_r1, W_l2, W_r2, W_o)


def _k5_body(s2_ref, cnt_ref, z_ref, bl2_ref, wo_ref, bo_ref, out_ref):
    inv = 1.0 / jnp.maximum(cnt_ref[0, :], 1.0)
    c0 = jnp.sum(bl2_ref[...] * wo_ref[...]) + bo_ref[0, 0]
    out_ref[...] = s2_ref[...] * inv[None, :] + z_ref[...] + c0


def _k5_combine(s2, cnt, z, b_l2, W_o, b_o):
    return pl.pallas_call(
        _k5_body,
        grid=(1,),
        in_specs=[
            pl.BlockSpec((B, N), lambda i: (0, 0)),
            pl.BlockSpec((1, N), lambda i: (0, 0)),
            pl.BlockSpec((B, N), lambda i: (0, 0)),
            pl.BlockSpec((1, G), lambda i: (0, 0)),
            pl.BlockSpec((1, G), lambda i: (0, 0)),
            pl.BlockSpec((1, 1), lambda i: (0, 0)),
        ],
        out_specs=pl.BlockSpec((B, N), lambda i: (0, 0)),
        out_shape=jax.ShapeDtypeStruct((B, N), jnp.float32),
    )(s2, cnt, z, b_l2, W_o, b_o)


def kernel(dynamic_features, static_features, edge_index, W_ih, W_hh, b_ih,
           b_hh, W_s, b_s, W_f, b_f, W_l1, b_l1, W_r1, W_l2, b_l2, W_r2, W_o,
           b_o):
    src = edge_index[0]
    dst = edge_index[1]
    npad = EPAD - E
    pad_src = (jnp.arange(npad, dtype=jnp.int32) * 37) % N
    pad_dst = N + (jnp.arange(npad, dtype=jnp.int32) % (NP - N))
    src2d = jnp.concatenate([src, pad_src]).reshape(EPAD // 128, 128)
    dst2d = jnp.concatenate([dst, pad_dst]).reshape(EPAD // 128, 128)
    src3 = jnp.stack([src2d, src2d + N])

    cnt = _k0_counts(dst2d)
    cnt_n = cnt[:N]
    cnt_col = jnp.concatenate([cnt_n, cnt_n]).reshape(B * N, 1)

    embed = _k1_encode(dynamic_features, static_features, W_ih, W_hh,
                       b_ih.reshape(1, 4 * H), b_hh.reshape(1, 4 * H), W_s,
                       b_s.reshape(1, H), W_f, b_f.reshape(1, H))

    sum1 = _k2_seg64(embed, src3, dst2d)

    y1, z1 = _k3_sage1(sum1, embed, cnt_col, W_l1, b_l1.reshape(1, G), W_r1,
                       W_l2, W_r2, W_o)

    s2 = _k4_seg1(y1.reshape(B * N), src3, dst2d).reshape(B, N)

    z = z1.reshape(B, N)
    pred = _k5_combine(s2, cnt_n.reshape(1, N), z, b_l2.reshape(1, G), W_o,
                       b_o.reshape(1, 1))
    return pred


# speedup vs baseline: 1.3280x; 1.1045x over previous
import functools

import jax
import jax.numpy as jnp
from jax import lax
from jax.experimental import pallas as pl
from jax.experimental.pallas import tpu as pltpu
from jax.experimental.pallas import tpu_sc as plsc

B, T, N, F_DYN, F_STA = 2, 12, 10000, 16, 16
H, G, E = 64, 64, 320000

NTILE = 16
NCORE = 2
EPAD = 327680
UNIT = 512
UPT = EPAD // (NTILE * UNIT)
NP = 10240
RPT = NP // NTILE
GP = 64
FW = T * F_DYN + F_STA


def _vmesh():
    return plsc.VectorSubcoreMesh(core_axis_name="c", subcore_axis_name="s")


def _zero_vmem_2d(buf, rows, cols):
    zv = jnp.zeros((16,), jnp.float32)

    @pl.loop(0, rows)
    def _(r):
        @pl.loop(0, cols // 16)
        def _(j):
            buf[r, pl.ds(j * 16, 16)] = zv


def _zero_vmem_1d(buf, n):
    zv = jnp.zeros((16,), jnp.float32)

    @pl.loop(0, n // 16)
    def _(j):
        buf[pl.ds(j * 16, 16)] = zv


def _k0_counts(dst2d):
    @functools.partial(
        pl.kernel,
        mesh=_vmesh(),
        compiler_params=pltpu.CompilerParams(use_tc_tiling_on_sc=False),
        out_type=jax.ShapeDtypeStruct((NCORE * NP,), jnp.float32),
        scratch_types=[
            pltpu.VMEM((80, 128), jnp.int32),
            pltpu.VMEM((128,), jnp.float32),
            pltpu.VMEM((RPT,), jnp.float32),
            pltpu.VMEM_SHARED((NP,), jnp.float32),
            pltpu.SemaphoreType.DMA,
            pltpu.SemaphoreType.DMA,
        ],
    )
    def k0(dst_hbm, out_hbm, didx, ones_v, zbuf, cnt_sh, semi, sems):
        c = lax.axis_index("c")
        s = lax.axis_index("s")
        ov = jnp.ones((16,), jnp.float32)

        row0 = (c * NTILE + s) * 80
        hi = pltpu.async_copy(dst_hbm.at[pl.ds(row0, 80)], didx, semi)

        @pl.loop(0, 8)
        def _(j):
            ones_v[pl.ds(j * 16, 16)] = ov

        _zero_vmem_1d(zbuf, RPT)
        pltpu.sync_copy(zbuf, cnt_sh.at[pl.ds(s * RPT, RPT)])
        hi.wait()
        plsc.subcore_barrier()

        @pl.loop(0, 80)
        def _(r):
            pltpu.async_copy(ones_v, cnt_sh.at[didx.at[r]], sems, add=True)

        @pl.loop(0, 80)
        def _(r):
            pltpu.make_async_copy(ones_v, cnt_sh.at[pl.ds(0, 128)],
                                  sems).wait()

        plsc.subcore_barrier()
        pltpu.sync_copy(cnt_sh.at[pl.ds(s * RPT, RPT)],
                        out_hbm.at[pl.ds(c * NP + s * RPT, RPT)])

    return k0(dst2d)


def _k2_seg64(x, src3, dst2d):
    @functools.partial(
        pl.kernel,
        mesh=_vmesh(),
        compiler_params=pltpu.CompilerParams(use_tc_tiling_on_sc=False),
        out_type=jax.ShapeDtypeStruct((B * N, GP), jnp.float32),
        scratch_types=[
            pltpu.VMEM((4, 4, 128), jnp.int32),
            pltpu.VMEM((4, 4, 128), jnp.int32),
            pltpu.VMEM((2, UNIT, GP), jnp.float32),
            pltpu.VMEM((128, GP), jnp.float32),
            pltpu.VMEM_SHARED((NP, GP), jnp.float32),
            pltpu.SemaphoreType.DMA,
            pltpu.SemaphoreType.DMA,
            pltpu.SemaphoreType.DMA,
            pltpu.SemaphoreType.DMA,
            pltpu.SemaphoreType.DMA,
            pltpu.SemaphoreType.DMA,
            pltpu.SemaphoreType.DMA,
            pltpu.SemaphoreType.DMA,
        ],
    )
    def k2(x_hbm, src_hbm, dst_hbm, out_hbm, sidx, didx, vals, zbuf, acc_sh,
           semi0, semi1, semi2, semi3, semg0, semg1, sems0, sems1):
        c = lax.axis_index("c")
        s = lax.axis_index("s")
        semi = (semi0, semi1, semi2, semi3)
        semg = (semg0, semg1)
        sems = (sems0, sems1)

        row0 = s * (UPT * 4)

        def fire_idx(si, u):
            r = row0 + jnp.minimum(u, UPT - 1) * 4
            pltpu.async_copy(src_hbm.at[c, pl.ds(r, 4)], sidx.at[si],
                             semi[si])
            pltpu.async_copy(dst_hbm.at[pl.ds(r, 4)], didx.at[si], semi[si])

        def wait_idx(si):
            pltpu.make_async_copy(dst_hbm.at[pl.ds(0, 4)], sidx.at[si],
                                  semi[si]).wait()
            pltpu.make_async_copy(dst_hbm.at[pl.ds(0, 4)], didx.at[si],
                                  semi[si]).wait()

        def fire_g(sv, si):
            for j in range(4):
                pltpu.async_copy(x_hbm.at[sidx.at[si, j]],
                                 vals.at[sv, pl.ds(j * 128, 128)], semg[sv])

        def wait_g(sv):
            for j in range(4):
                pltpu.make_async_copy(x_hbm.at[pl.ds(0, 128)],
                                      vals.at[sv, pl.ds(j * 128, 128)],
                                      semg[sv]).wait()

        def fire_s(sv, si):
            for j in range(4):
                pltpu.async_copy(vals.at[sv, pl.ds(j * 128, 128)],
                                 acc_sh.at[didx.at[si, j]], sems[sv],
                                 add=True)

        def wait_s(sv):
            for j in range(4):
                pltpu.make_async_copy(vals.at[sv, pl.ds(j * 128, 128)],
                                      acc_sh.at[pl.ds(0, 128)],
                                      sems[sv]).wait()

        fire_idx(0, 0)

        _zero_vmem_2d(zbuf, 128, GP)

        @pl.loop(0, RPT // 128)
        def _(k):
            pltpu.sync_copy(zbuf, acc_sh.at[pl.ds(s * RPT + k * 128, 128)])

        plsc.subcore_barrier()

        for u in range(4):
            if u >= 2:
                wait_s(u % 2)
            fire_idx((u + 1) % 4, u + 1)
            wait_idx(u % 4)
            fire_g(u % 2, u % 4)
            wait_g(u % 2)
            fire_s(u % 2, u % 4)

        @pl.loop(0, (UPT - 4) // 4)
        def _(pp):
            for q in range(4):
                u = 4 + pp * 4 + q
                wait_s(q % 2)
                fire_idx((q + 1) % 4, u + 1)
                wait_idx(q)
                fire_g(q % 2, q)
                wait_g(q % 2)
                fire_s(q % 2, q)

        wait_idx(0)
        wait_s(0)
        wait_s(1)

        plsc.subcore_barrier()
        last = (NTILE - 1) * RPT

        @pl.when(s < NTILE - 1)
        def _():
            pltpu.sync_copy(acc_sh.at[pl.ds(s * RPT, RPT)],
                            out_hbm.at[pl.ds(c * N + s * RPT, RPT)])

        @pl.when(s == NTILE - 1)
        def _():
            pltpu.sync_copy(acc_sh.at[pl.ds(last, N - last)],
                            out_hbm.at[pl.ds(c * N + last, N - last)])

    return k2(x, src3, dst2d)


U4 = 2048
UPT4 = EPAD // (NTILE * U4)


def _k4_seg1(y, src3, dst2d):
    @functools.partial(
        pl.kernel,
        mesh=_vmesh(),
        compiler_params=pltpu.CompilerParams(use_tc_tiling_on_sc=False),
        out_type=jax.ShapeDtypeStruct((B * N,), jnp.float32),
        scratch_types=[
            pltpu.VMEM((UPT4 * 16, 128), jnp.int32),
            pltpu.VMEM((UPT4 * 16, 128), jnp.int32),
            pltpu.VMEM((2, U4), jnp.float32),
            pltpu.VMEM((RPT,), jnp.float32),
            pltpu.VMEM_SHARED((NP,), jnp.float32),
            pltpu.SemaphoreType.DMA,
            pltpu.SemaphoreType.DMA,
            pltpu.SemaphoreType.DMA,
            pltpu.SemaphoreType.DMA,
            pltpu.SemaphoreType.DMA,
        ],
    )
    def k4(y_hbm, src_hbm, dst_hbm, out_hbm, sidx, didx, vals, zbuf, acc_sh,
           semi, semg0, semg1, sems0, sems1):
        c = lax.axis_index("c")
        s = lax.axis_index("s")
        semg = (semg0, semg1)
        sems = (sems0, sems1)

        row0 = s * (UPT4 * 16)
        h1 = pltpu.async_copy(src_hbm.at[c, pl.ds(row0, UPT4 * 16)], sidx,
                              semi)
        h2 = pltpu.async_copy(dst_hbm.at[pl.ds(row0, UPT4 * 16)], didx, semi)

        _zero_vmem_1d(zbuf, RPT)
        pltpu.sync_copy(zbuf, acc_sh.at[pl.ds(s * RPT, RPT)])
        h1.wait()
        h2.wait()
        plsc.subcore_barrier()

        def fire_g(slot, u):
            for j in range(16):
                pltpu.async_copy(y_hbm.at[sidx.at[u * 16 + j]],
                                 vals.at[slot, pl.ds(j * 128, 128)],
                                 semg[slot])

        def wait_g(slot):
            for j in range(16):
                pltpu.make_async_copy(y_hbm.at[pl.ds(0, 128)],
                                      vals.at[slot, pl.ds(j * 128, 128)],
                                      semg[slot]).wait()

        def fire_s(slot, u):
            for j in range(16):
                pltpu.async_copy(vals.at[slot, pl.ds(j * 128, 128)],
                                 acc_sh.at[didx.at[u * 16 + j]], sems[slot],
                                 add=True)

        def wait_s(slot):
            for j in range(16):
                pltpu.make_async_copy(vals.at[slot, pl.ds(j * 128, 128)],
                                      acc_sh.at[pl.ds(0, 128)],
                                      sems[slot]).wait()

        for slot in (0, 1):
            fire_g(slot, slot)
            wait_g(slot)
            fire_s(slot, slot)

        @pl.loop(0, (UPT4 - 2) // 2)
        def _(p):
            u = 2 + p * 2
            for slot in (0, 1):
                wait_s(slot)
                fire_g(slot, u + slot)
                wait_g(slot)
                fire_s(slot, u + slot)

        wait_s(0)
        wait_s(1)

        plsc.subcore_barrier()
        last = (NTILE - 1) * RPT

        @pl.when(s < NTILE - 1)
        def _():
            pltpu.sync_copy(acc_sh.at[pl.ds(s * RPT, RPT)],
                            out_hbm.at[pl.ds(c * N + s * RPT, RPT)])

        @pl.when(s == NTILE - 1)
        def _():
            pltpu.sync_copy(acc_sh.at[pl.ds(last, N - last)],
                            out_hbm.at[pl.ds(c * N + last, N - last)])

    return k4(y, src3, dst2d)


def _k1_body(*refs):
    dyn_refs = refs[:T]
    sta_ref, wih_ref, whh_ref, bih_ref, bhh_ref, ws_ref, bs_ref, wf_ref, \
        bf_ref, out_ref = refs[T:]
    blk = out_ref.shape[0]
    dn = (((1,), (1,)), ((), ()))
    wih = wih_ref[...]
    whh = whh_ref[...]
    bias = bih_ref[...] + bhh_ref[...]
    h = None
    c = None
    for t in range(T):
        x_t = dyn_refs[t][0, 0]
        gates = lax.dot_general(x_t, wih, dn,
                                preferred_element_type=jnp.float32) + bias
        if h is not None:
            gates = gates + lax.dot_general(h, whh, dn,
                                            preferred_element_type=jnp.float32)
        gi = jax.nn.sigmoid(gates[:, 0 * H:1 * H])
        gf = jax.nn.sigmoid(gates[:, 1 * H:2 * H])
        gg = jnp.tanh(gates[:, 2 * H:3 * H])
        go = jax.nn.sigmoid(gates[:, 3 * H:4 * H])
        c = gi * gg if c is None else gf * c + gi * gg
        h = go * jnp.tanh(c)
    s_t = jax.nn.relu(
        lax.dot_general(sta_ref[0], ws_ref[...], dn,
                        preferred_element_type=jnp.float32) + bs_ref[...])
    wf = wf_ref[...]
    emb = jax.nn.relu(
        lax.dot_general(h, wf[:, :H], dn, preferred_element_type=jnp.float32)
        + lax.dot_general(s_t, wf[:, H:], dn,
                          preferred_element_type=jnp.float32)
        + bf_ref[...])
    out_ref[...] = emb


def _k1_encode(dyn, sta, W_ih, W_hh, b_ih, b_hh, W_s, b_s, W_f, b_f):
    blk = 1000
    nb = N // blk
    grid = (B * nb,)
    dyn_specs = [
        pl.BlockSpec((1, 1, blk, F_DYN), lambda i, t=t: (i // nb, t, i % nb, 0))
        for t in range(T)
    ]
    return pl.pallas_call(
        _k1_body,
        grid=grid,
        in_specs=dyn_specs + [
            pl.BlockSpec((1, blk, F_STA), lambda i: (i // nb, i % nb, 0)),
            pl.BlockSpec((4 * H, F_DYN), lambda i: (0, 0)),
            pl.BlockSpec((4 * H, H), lambda i: (0, 0)),
            pl.BlockSpec((1, 4 * H), lambda i: (0, 0)),
            pl.BlockSpec((1, 4 * H), lambda i: (0, 0)),
            pl.BlockSpec((H, F_STA), lambda i: (0, 0)),
            pl.BlockSpec((1, H), lambda i: (0, 0)),
            pl.BlockSpec((H, 2 * H), lambda i: (0, 0)),
            pl.BlockSpec((1, H), lambda i: (0, 0)),
        ],
        out_specs=pl.BlockSpec((blk, H), lambda i: (i, 0)),
        out_shape=jax.ShapeDtypeStruct((B * N, H), jnp.float32),
    )(*([dyn] * T), sta, W_ih, W_hh, b_ih, b_hh, W_s, b_s, W_f, b_f)


def _k3_body(sum1_ref, x_ref, cnt_ref, wl1_ref, bl1_ref, wr1_ref, wl2_ref,
             wr2_ref, wo_ref, y_ref, z_ref):
    inv = 1.0 / jnp.maximum(cnt_ref[...], 1.0)
    mean = sum1_ref[:, :G] * inv
    dn = (((1,), (1,)), ((), ()))
    x1 = jax.nn.relu(
        lax.dot_general(mean, wl1_ref[...], dn, preferred_element_type=jnp.float32)
        + lax.dot_general(x_ref[:, :G], wr1_ref[...], dn,
                          preferred_element_type=jnp.float32)
        + bl1_ref[...])
    wo = wo_ref[...]
    vl = lax.dot_general(wo, wl2_ref[...], (((1,), (0,)), ((), ())),
                         preferred_element_type=jnp.float32)
    vr = lax.dot_general(wo, wr2_ref[...], (((1,), (0,)), ((), ())),
                         preferred_element_type=jnp.float32)
    y_ref[...] = jnp.sum(x1 * vl, axis=1, keepdims=True)
    z_ref[...] = jnp.sum(x1 * vr, axis=1, keepdims=True)


def _k3_sage1(sum1, embed, cnt_col, W_l1, b_l1, W_r1, W_l2, W_r2, W_o):
    blk = 2000
    grid = (B * N // blk,)
    return pl.pallas_call(
        _k3_body,
        grid=grid,
        in_specs=[
            pl.BlockSpec((blk, GP), lambda i: (i, 0)),
            pl.BlockSpec((blk, GP), lambda i: (i, 0)),
            pl.BlockSpec((blk, 1), lambda i: (i, 0)),
            pl.BlockSpec((G, G), lambda i: (0, 0)),
            pl.BlockSpec((1, G), lambda i: (0, 0)),
            pl.BlockSpec((G, G), lambda i: (0, 0)),
            pl.BlockSpec((G, G), lambda i: (0, 0)),
            pl.BlockSpec((G, G), lambda i: (0, 0)),
            pl.BlockSpec((1, G), lambda i: (0, 0)),
        ],
        out_specs=[
            pl.BlockSpec((blk, 1), lambda i: (i, 0)),
            pl.BlockSpec((blk, 1), lambda i: (i, 0)),
        ],
        out_shape=[
            jax.ShapeDtypeStruct((B * N, 1), jnp.float32),
            jax.ShapeDtypeStruct((B * N, 1), jnp.float32),
        ],
    )(sum1, embed, cnt_col, W_l1, b_l1, W_r1, W_l2, W_r2, W_o)


def _k5_body(s2_ref, cnt_ref, z_ref, bl2_ref, wo_ref, bo_ref, out_ref):
    inv = 1.0 / jnp.maximum(cnt_ref[0, :], 1.0)
    c0 = jnp.sum(bl2_ref[...] * wo_ref[...]) + bo_ref[0, 0]
    out_ref[...] = s2_ref[...] * inv[None, :] + z_ref[...] + c0


def _k5_combine(s2, cnt, z, b_l2, W_o, b_o):
    return pl.pallas_call(
        _k5_body,
        grid=(1,),
        in_specs=[
            pl.BlockSpec((B, N), lambda i: (0, 0)),
            pl.BlockSpec((1, N), lambda i: (0, 0)),
            pl.BlockSpec((B, N), lambda i: (0, 0)),
            pl.BlockSpec((1, G), lambda i: (0, 0)),
            pl.BlockSpec((1, G), lambda i: (0, 0)),
            pl.BlockSpec((1, 1), lambda i: (0, 0)),
        ],
        out_specs=pl.BlockSpec((B, N), lambda i: (0, 0)),
        out_shape=jax.ShapeDtypeStruct((B, N), jnp.float32),
    )(s2, cnt, z, b_l2, W_o, b_o)


def kernel(dynamic_features, static_features, edge_index, W_ih, W_hh, b_ih,
           b_hh, W_s, b_s, W_f, b_f, W_l1, b_l1, W_r1, W_l2, b_l2, W_r2, W_o,
           b_o):
    src = edge_index[0]
    dst = edge_index[1]
    npad = EPAD - E
    pad_src = (jnp.arange(npad, dtype=jnp.int32) * 37) % N
    pad_dst = N + (jnp.arange(npad, dtype=jnp.int32) % (NP - N))
    src2d = jnp.concatenate([src, pad_src]).reshape(EPAD // 128, 128)
    dst2d = jnp.concatenate([dst, pad_dst]).reshape(EPAD // 128, 128)
    src3 = jnp.stack([src2d, src2d + N])

    cnt_p = _k0_counts(dst2d)
    cnt_n = cnt_p[:N] + cnt_p[NP:NP + N]
    cnt_col = jnp.concatenate([cnt_n, cnt_n]).reshape(B * N, 1)

    embed = _k1_encode(dynamic_features, static_features, W_ih, W_hh,
                       b_ih.reshape(1, 4 * H), b_hh.reshape(1, 4 * H), W_s,
                       b_s.reshape(1, H), W_f, b_f.reshape(1, H))

    sum1 = _k2_seg64(embed, src3, dst2d)

    y1, z1 = _k3_sage1(sum1, embed, cnt_col, W_l1, b_l1.reshape(1, G), W_r1,
                       W_l2, W_r2, W_o)

    s2 = _k4_seg1(y1.reshape(B * N), src3, dst2d).reshape(B, N)

    z = z1.reshape(B, N)
    pred = _k5_combine(s2, cnt_n.reshape(1, N), z, b_l2.reshape(1, G), W_o,
                       b_o.reshape(1, 1))
    return pred


# speedup vs baseline: 1.3288x; 1.0006x over previous
import functools

import jax
import jax.numpy as jnp
from jax import lax
from jax.experimental import pallas as pl
from jax.experimental.pallas import tpu as pltpu
from jax.experimental.pallas import tpu_sc as plsc

B, T, N, F_DYN, F_STA = 2, 12, 10000, 16, 16
H, G, E = 64, 64, 320000

NTILE = 16
NCORE = 2
EPAD = 327680
UNIT = 512
UPT = EPAD // (NTILE * UNIT)
NP = 10240
RPT = NP // NTILE
GP = 64


def _vmesh():
    return plsc.VectorSubcoreMesh(core_axis_name="c", subcore_axis_name="s")


def _zero_vmem_2d(buf, rows, cols):
    zv = jnp.zeros((16,), jnp.float32)

    @pl.loop(0, rows)
    def _(r):
        @pl.loop(0, cols // 16)
        def _(j):
            buf[r, pl.ds(j * 16, 16)] = zv


def _zero_vmem_1d(buf, n):
    zv = jnp.zeros((16,), jnp.float32)

    @pl.loop(0, n // 16)
    def _(j):
        buf[pl.ds(j * 16, 16)] = zv


def _k0_counts(dst2d):
    @functools.partial(
        pl.kernel,
        mesh=_vmesh(),
        compiler_params=pltpu.CompilerParams(use_tc_tiling_on_sc=False),
        out_type=jax.ShapeDtypeStruct((NCORE * NP,), jnp.float32),
        scratch_types=[
            pltpu.VMEM((80, 128), jnp.int32),
            pltpu.VMEM((128,), jnp.float32),
            pltpu.VMEM((RPT,), jnp.float32),
            pltpu.VMEM_SHARED((NP,), jnp.float32),
            pltpu.SemaphoreType.DMA,
            pltpu.SemaphoreType.DMA,
        ],
    )
    def k0(dst_hbm, out_hbm, didx, ones_v, zbuf, cnt_sh, semi, sems):
        c = lax.axis_index("c")
        s = lax.axis_index("s")
        ov = jnp.ones((16,), jnp.float32)

        row0 = (c * NTILE + s) * 80
        hi = pltpu.async_copy(dst_hbm.at[pl.ds(row0, 80)], didx, semi)

        @pl.loop(0, 8)
        def _(j):
            ones_v[pl.ds(j * 16, 16)] = ov

        _zero_vmem_1d(zbuf, RPT)
        pltpu.sync_copy(zbuf, cnt_sh.at[pl.ds(s * RPT, RPT)])
        hi.wait()
        plsc.subcore_barrier()

        @pl.loop(0, 80)
        def _(r):
            pltpu.async_copy(ones_v, cnt_sh.at[didx.at[r]], sems, add=True)

        @pl.loop(0, 80)
        def _(r):
            pltpu.make_async_copy(ones_v, cnt_sh.at[pl.ds(0, 128)],
                                  sems).wait()

        plsc.subcore_barrier()
        pltpu.sync_copy(cnt_sh.at[pl.ds(s * RPT, RPT)],
                        out_hbm.at[pl.ds(c * NP + s * RPT, RPT)])

    return k0(dst2d)


def _k2_seg64(x, src3, dst2d):
    @functools.partial(
        pl.kernel,
        mesh=_vmesh(),
        compiler_params=pltpu.CompilerParams(use_tc_tiling_on_sc=False),
        out_type=jax.ShapeDtypeStruct((B * N, GP), jnp.float32),
        scratch_types=[
            pltpu.VMEM((4, 4, 128), jnp.int32),
            pltpu.VMEM((4, 4, 128), jnp.int32),
            pltpu.VMEM((2, UNIT, GP), jnp.float32),
            pltpu.VMEM((128, GP), jnp.float32),
            pltpu.VMEM_SHARED((NP, GP), jnp.float32),
            pltpu.SemaphoreType.DMA,
            pltpu.SemaphoreType.DMA,
            pltpu.SemaphoreType.DMA,
            pltpu.SemaphoreType.DMA,
            pltpu.SemaphoreType.DMA,
            pltpu.SemaphoreType.DMA,
            pltpu.SemaphoreType.DMA,
            pltpu.SemaphoreType.DMA,
        ],
    )
    def k2(x_hbm, src_hbm, dst_hbm, out_hbm, sidx, didx, vals, zbuf, acc_sh,
           semi0, semi1, semi2, semi3, semg0, semg1, sems0, sems1):
        c = lax.axis_index("c")
        s = lax.axis_index("s")
        semi = (semi0, semi1, semi2, semi3)
        semg = (semg0, semg1)
        sems = (sems0, sems1)

        row0 = s * (UPT * 4)

        def fire_idx(si, u):
            r = row0 + jnp.minimum(u, UPT - 1) * 4
            pltpu.async_copy(src_hbm.at[c, pl.ds(r, 4)], sidx.at[si],
                             semi[si])
            pltpu.async_copy(dst_hbm.at[pl.ds(r, 4)], didx.at[si], semi[si])

        def wait_idx(si):
            pltpu.make_async_copy(dst_hbm.at[pl.ds(0, 4)], sidx.at[si],
                                  semi[si]).wait()
            pltpu.make_async_copy(dst_hbm.at[pl.ds(0, 4)], didx.at[si],
                                  semi[si]).wait()

        def fire_g(sv, si):
            for j in range(4):
                pltpu.async_copy(x_hbm.at[sidx.at[si, j]],
                                 vals.at[sv, pl.ds(j * 128, 128)], semg[sv])

        def wait_g(sv):
            for j in range(4):
                pltpu.make_async_copy(x_hbm.at[pl.ds(0, 128)],
                                      vals.at[sv, pl.ds(j * 128, 128)],
                                      semg[sv]).wait()

        def fire_s(sv, si):
            for j in range(4):
                pltpu.async_copy(vals.at[sv, pl.ds(j * 128, 128)],
                                 acc_sh.at[didx.at[si, j]], sems[sv],
                                 add=True)

        def wait_s(sv):
            for j in range(4):
                pltpu.make_async_copy(vals.at[sv, pl.ds(j * 128, 128)],
                                      acc_sh.at[pl.ds(0, 128)],
                                      sems[sv]).wait()

        fire_idx(0, 0)

        _zero_vmem_2d(zbuf, 128, GP)

        @pl.loop(0, RPT // 128)
        def _(k):
            pltpu.sync_copy(zbuf, acc_sh.at[pl.ds(s * RPT + k * 128, 128)])

        plsc.subcore_barrier()

        for u in range(4):
            if u >= 2:
                wait_s(u % 2)
            fire_idx((u + 1) % 4, u + 1)
            wait_idx(u % 4)
            fire_g(u % 2, u % 4)
            wait_g(u % 2)
            fire_s(u % 2, u % 4)

        @pl.loop(0, (UPT - 4) // 4)
        def _(pp):
            for q in range(4):
                u = 4 + pp * 4 + q
                wait_s(q % 2)
                fire_idx((q + 1) % 4, u + 1)
                wait_idx(q)
                fire_g(q % 2, q)
                wait_g(q % 2)
                fire_s(q % 2, q)

        wait_idx(0)
        wait_s(0)
        wait_s(1)

        plsc.subcore_barrier()
        last = (NTILE - 1) * RPT

        @pl.when(s < NTILE - 1)
        def _():
            pltpu.sync_copy(acc_sh.at[pl.ds(s * RPT, RPT)],
                            out_hbm.at[pl.ds(c * N + s * RPT, RPT)])

        @pl.when(s == NTILE - 1)
        def _():
            pltpu.sync_copy(acc_sh.at[pl.ds(last, N - last)],
                            out_hbm.at[pl.ds(c * N + last, N - last)])

    return k2(x, src3, dst2d)


U4 = 2048
UPT4 = EPAD // (NTILE * U4)


def _k4_seg1(y, src3, dst2d):
    @functools.partial(
        pl.kernel,
        mesh=_vmesh(),
        compiler_params=pltpu.CompilerParams(use_tc_tiling_on_sc=False),
        out_type=jax.ShapeDtypeStruct((B * N,), jnp.float32),
        scratch_types=[
            pltpu.VMEM((UPT4 * 16, 128), jnp.int32),
            pltpu.VMEM((UPT4 * 16, 128), jnp.int32),
            pltpu.VMEM((2, U4), jnp.float32),
            pltpu.VMEM((RPT,), jnp.float32),
            pltpu.VMEM_SHARED((NP,), jnp.float32),
            pltpu.SemaphoreType.DMA,
            pltpu.SemaphoreType.DMA,
            pltpu.SemaphoreType.DMA,
            pltpu.SemaphoreType.DMA,
            pltpu.SemaphoreType.DMA,
        ],
    )
    def k4(y_hbm, src_hbm, dst_hbm, out_hbm, sidx, didx, vals, zbuf, acc_sh,
           semi, semg0, semg1, sems0, sems1):
        c = lax.axis_index("c")
        s = lax.axis_index("s")
        semg = (semg0, semg1)
        sems = (sems0, sems1)

        row0 = s * (UPT4 * 16)
        h1 = pltpu.async_copy(src_hbm.at[c, pl.ds(row0, UPT4 * 16)], sidx,
                              semi)
        h2 = pltpu.async_copy(dst_hbm.at[pl.ds(row0, UPT4 * 16)], didx, semi)

        _zero_vmem_1d(zbuf, RPT)
        pltpu.sync_copy(zbuf, acc_sh.at[pl.ds(s * RPT, RPT)])
        h1.wait()
        h2.wait()
        plsc.subcore_barrier()

        def fire_g(slot, u):
            for j in range(16):
                pltpu.async_copy(y_hbm.at[sidx.at[u * 16 + j]],
                                 vals.at[slot, pl.ds(j * 128, 128)],
                                 semg[slot])

        def wait_g(slot):
            for j in range(16):
                pltpu.make_async_copy(y_hbm.at[pl.ds(0, 128)],
                                      vals.at[slot, pl.ds(j * 128, 128)],
                                      semg[slot]).wait()

        def fire_s(slot, u):
            for j in range(16):
                pltpu.async_copy(vals.at[slot, pl.ds(j * 128, 128)],
                                 acc_sh.at[didx.at[u * 16 + j]], sems[slot],
                                 add=True)

        def wait_s(slot):
            for j in range(16):
                pltpu.make_async_copy(vals.at[slot, pl.ds(j * 128, 128)],
                                      acc_sh.at[pl.ds(0, 128)],
                                      sems[slot]).wait()

        for slot in (0, 1):
            fire_g(slot, slot)
            wait_g(slot)
            fire_s(slot, slot)

        @pl.loop(0, (UPT4 - 2) // 2)
        def _(p):
            u = 2 + p * 2
            for slot in (0, 1):
                wait_s(slot)
                fire_g(slot, u + slot)
                wait_g(slot)
                fire_s(slot, u + slot)

        wait_s(0)
        wait_s(1)

        plsc.subcore_barrier()
        last = (NTILE - 1) * RPT

        @pl.when(s < NTILE - 1)
        def _():
            pltpu.sync_copy(acc_sh.at[pl.ds(s * RPT, RPT)],
                            out_hbm.at[pl.ds(c * N + s * RPT, RPT)])

        @pl.when(s == NTILE - 1)
        def _():
            pltpu.sync_copy(acc_sh.at[pl.ds(last, N - last)],
                            out_hbm.at[pl.ds(c * N + last, N - last)])

    return k4(y, src3, dst2d)


def _k1_body(*refs):
    dyn_refs = refs[:T]
    sta_ref, wih_ref, whh_ref, bih_ref, bhh_ref, ws_ref, bs_ref, wf_ref, \
        bf_ref, out_ref = refs[T:]
    blk = out_ref.shape[0]
    dn = (((1,), (1,)), ((), ()))
    wih = wih_ref[...]
    whh = whh_ref[...]
    bias = bih_ref[...] + bhh_ref[...]
    h = None
    c = None
    for t in range(T):
        x_t = dyn_refs[t][0, 0]
        gates = lax.dot_general(x_t, wih, dn,
                                preferred_element_type=jnp.float32) + bias
        if h is not None:
            gates = gates + lax.dot_general(h, whh, dn,
                                            preferred_element_type=jnp.float32)
        gi = jax.nn.sigmoid(gates[:, 0 * H:1 * H])
        gf = jax.nn.sigmoid(gates[:, 1 * H:2 * H])
        gg = jnp.tanh(gates[:, 2 * H:3 * H])
        go = jax.nn.sigmoid(gates[:, 3 * H:4 * H])
        c = gi * gg if c is None else gf * c + gi * gg
        h = go * jnp.tanh(c)
    s_t = jax.nn.relu(
        lax.dot_general(sta_ref[0], ws_ref[...], dn,
                        preferred_element_type=jnp.float32) + bs_ref[...])
    wf = wf_ref[...]
    emb = jax.nn.relu(
        lax.dot_general(h, wf[:, :H], dn, preferred_element_type=jnp.float32)
        + lax.dot_general(s_t, wf[:, H:], dn,
                          preferred_element_type=jnp.float32)
        + bf_ref[...])
    out_ref[...] = emb


def _k1_encode(dyn, sta, W_ih, W_hh, b_ih, b_hh, W_s, b_s, W_f, b_f):
    blk = 1000
    nb = N // blk
    grid = (B * nb,)
    dyn_specs = [
        pl.BlockSpec((1, 1, blk, F_DYN), lambda i, t=t: (i // nb, t, i % nb, 0))
        for t in range(T)
    ]
    return pl.pallas_call(
        _k1_body,
        grid=grid,
        in_specs=dyn_specs + [
            pl.BlockSpec((1, blk, F_STA), lambda i: (i // nb, i % nb, 0)),
            pl.BlockSpec((4 * H, F_DYN), lambda i: (0, 0)),
            pl.BlockSpec((4 * H, H), lambda i: (0, 0)),
            pl.BlockSpec((1, 4 * H), lambda i: (0, 0)),
            pl.BlockSpec((1, 4 * H), lambda i: (0, 0)),
            pl.BlockSpec((H, F_STA), lambda i: (0, 0)),
            pl.BlockSpec((1, H), lambda i: (0, 0)),
            pl.BlockSpec((H, 2 * H), lambda i: (0, 0)),
            pl.BlockSpec((1, H), lambda i: (0, 0)),
        ],
        out_specs=pl.BlockSpec((blk, H), lambda i: (i, 0)),
        out_shape=jax.ShapeDtypeStruct((B * N, H), jnp.float32),
    )(*([dyn] * T), sta, W_ih, W_hh, b_ih, b_hh, W_s, b_s, W_f, b_f)


def _k3_body(sum1_ref, x_ref, cnt_ref, wl1_ref, bl1_ref, wr1_ref, wl2_ref,
             wr2_ref, wo_ref, y_ref, z_ref):
    inv = 1.0 / jnp.maximum(cnt_ref[...], 1.0)
    mean = sum1_ref[:, :G] * inv
    dn = (((1,), (1,)), ((), ()))
    x1 = jax.nn.relu(
        lax.dot_general(mean, wl1_ref[...], dn, preferred_element_type=jnp.float32)
        + lax.dot_general(x_ref[:, :G], wr1_ref[...], dn,
                          preferred_element_type=jnp.float32)
        + bl1_ref[...])
    wo = wo_ref[...]
    vl = lax.dot_general(wo, wl2_ref[...], (((1,), (0,)), ((), ())),
                         preferred_element_type=jnp.float32)
    vr = lax.dot_general(wo, wr2_ref[...], (((1,), (0,)), ((), ())),
                         preferred_element_type=jnp.float32)
    y_ref[...] = jnp.sum(x1 * vl, axis=1, keepdims=True)
    z_ref[...] = jnp.sum(x1 * vr, axis=1, keepdims=True)


def _k3_sage1(sum1, embed, cnt_col, W_l1, b_l1, W_r1, W_l2, W_r2, W_o):
    blk = 2000
    grid = (B * N // blk,)
    return pl.pallas_call(
        _k3_body,
        grid=grid,
        in_specs=[
            pl.BlockSpec((blk, GP), lambda i: (i, 0)),
            pl.BlockSpec((blk, GP), lambda i: (i, 0)),
            pl.BlockSpec((blk, 1), lambda i: (i, 0)),
            pl.BlockSpec((G, G), lambda i: (0, 0)),
            pl.BlockSpec((1, G), lambda i: (0, 0)),
            pl.BlockSpec((G, G), lambda i: (0, 0)),
            pl.BlockSpec((G, G), lambda i: (0, 0)),
            pl.BlockSpec((G, G), lambda i: (0, 0)),
            pl.BlockSpec((1, G), lambda i: (0, 0)),
        ],
        out_specs=[
            pl.BlockSpec((blk, 1), lambda i: (i, 0)),
            pl.BlockSpec((blk, 1), lambda i: (i, 0)),
        ],
        out_shape=[
            jax.ShapeDtypeStruct((B * N, 1), jnp.float32),
            jax.ShapeDtypeStruct((B * N, 1), jnp.float32),
        ],
    )(sum1, embed, cnt_col, W_l1, b_l1, W_r1, W_l2, W_r2, W_o)


def _k5_body(s2_ref, cnt_ref, z_ref, bl2_ref, wo_ref, bo_ref, out_ref):
    inv = 1.0 / jnp.maximum(cnt_ref[0, :], 1.0)
    c0 = jnp.sum(bl2_ref[...] * wo_ref[...]) + bo_ref[0, 0]
    out_ref[...] = s2_ref[...] * inv[None, :] + z_ref[...] + c0


def _k5_combine(s2, cnt, z, b_l2, W_o, b_o):
    return pl.pallas_call(
        _k5_body,
        grid=(1,),
        in_specs=[
            pl.BlockSpec((B, N), lambda i: (0, 0)),
            pl.BlockSpec((1, N), lambda i: (0, 0)),
            pl.BlockSpec((B, N), lambda i: (0, 0)),
            pl.BlockSpec((1, G), lambda i: (0, 0)),
            pl.BlockSpec((1, G), lambda i: (0, 0)),
            pl.BlockSpec((1, 1), lambda i: (0, 0)),
        ],
        out_specs=pl.BlockSpec((B, N), lambda i: (0, 0)),
        out_shape=jax.ShapeDtypeStruct((B, N), jnp.float32),
    )(s2, cnt, z, b_l2, W_o, b_o)


def kernel(dynamic_features, static_features, edge_index, W_ih, W_hh, b_ih,
           b_hh, W_s, b_s, W_f, b_f, W_l1, b_l1, W_r1, W_l2, b_l2, W_r2, W_o,
           b_o):
    src = edge_index[0]
    dst = edge_index[1]
    npad = EPAD - E
    pad_src = (jnp.arange(npad, dtype=jnp.int32) * 37) % N
    pad_dst = N + (jnp.arange(npad, dtype=jnp.int32) % (NP - N))
    src2d = jnp.concatenate([src, pad_src]).reshape(EPAD // 128, 128)
    dst2d = jnp.concatenate([dst, pad_dst]).reshape(EPAD // 128, 128)
    src3 = jnp.stack([src2d, src2d + N])

    cnt_p = _k0_counts(dst2d)
    cnt_n = cnt_p[:N] + cnt_p[NP:NP + N]
    cnt_col = jnp.concatenate([cnt_n, cnt_n]).reshape(B * N, 1)

    embed = _k1_encode(dynamic_features, static_features, W_ih, W_hh,
                       b_ih.reshape(1, 4 * H), b_hh.reshape(1, 4 * H), W_s,
                       b_s.reshape(1, H), W_f, b_f.reshape(1, H))

    sum1 = _k2_seg64(embed, src3, dst2d)

    y1, z1 = _k3_sage1(sum1, embed, cnt_col, W_l1, b_l1.reshape(1, G), W_r1,
                       W_l2, W_r2, W_o)

    s2 = _k4_seg1(y1.reshape(B * N), src3, dst2d).reshape(B, N)

    z = z1.reshape(B, N)
    pred = _k5_combine(s2, cnt_n.reshape(1, N), z, b_l2.reshape(1, G), W_o,
                       b_o.reshape(1, 1))
    return pred
